# baseline (speedup 1.0000x reference)
import functools
import jax
import jax.numpy as jnp
from jax import lax
from jax.experimental import pallas as pl
from jax.experimental.pallas import tpu as pltpu
from jax.experimental.pallas import tpu_sc as plsc

NC = 2
NS = 16
NW = NC * NS
CHUNK = 50


def _sc_mesh():
    return plsc.VectorSubcoreMesh(core_axis_name="c", subcore_axis_name="s")


def _stripe_copy(sid, n, src_at, dst_at):
    r_small = (n // NS) // 8 * 8
    r_last = n - r_small * (NS - 1)

    @pl.when(sid < NS - 1)
    def _():
        pltpu.sync_copy(src_at(sid * r_small, r_small),
                        dst_at(sid * r_small, r_small))

    @pl.when(sid == NS - 1)
    def _():
        pltpu.sync_copy(src_at((NS - 1) * r_small, r_last),
                        dst_at((NS - 1) * r_small, r_last))


GK = 5
GKD = 10


@functools.partial(jax.jit, static_argnames=("n", "e", "d", "gather"))
def _edge_aggregate(src, dst, table, ones, zeros, *, n, e, d, gather=True):
    per_w = e // NW
    nchunks = per_w // CHUNK

    src3 = src.reshape(NW, nchunks, CHUNK)
    dst3 = dst.reshape(NW, nchunks, CHUNK)

    gk = GK if gather else GKD
    ngroups = nchunks // gk
    ntail = nchunks - ngroups * gk

    if gather:
        bufs_scratch = (
            [pltpu.VMEM((CHUNK, d), jnp.float32) for _ in range(gk)]
            + [pltpu.VMEM((CHUNK,), jnp.int32) for _ in range(2 * gk)]
            + [pltpu.SemaphoreType.DMA for _ in range(3 * gk)]
        )
    else:
        bufs_scratch = (
            [pltpu.VMEM((CHUNK, d), jnp.float32)]
            + [pltpu.VMEM((CHUNK,), jnp.int32) for _ in range(gk)]
            + [pltpu.SemaphoreType.DMA for _ in range(2 * gk)]
        )

    @functools.partial(
        pl.kernel,
        out_type=jax.ShapeDtypeStruct((NC, n, d), jnp.float32),
        mesh=_sc_mesh(),
        scratch_types=[pltpu.VMEM_SHARED((n, d), jnp.float32)] + bufs_scratch,
    )
    def k(src_hbm, dst_hbm, table_hbm, ones_hbm, z_hbm, out_hbm, agg_sh, *bufs):
        cid = lax.axis_index("c")
        sid = lax.axis_index("s")
        wid = cid * NS + sid
        _stripe_copy(sid, n, lambda o, s: z_hbm.at[pl.ds(o, s)],
                     lambda o, s: agg_sh.at[pl.ds(o, s)])

        if gather:
            rows = bufs[:gk]
            sidx = bufs[gk:2 * gk]
            didx = bufs[2 * gk:3 * gk]
            isem = bufs[3 * gk:4 * gk]
            gsem = bufs[4 * gk:5 * gk]
            ssem = bufs[5 * gk:6 * gk]
            plsc.subcore_barrier()

            def group(c0, m):
                iw = []
                for t in range(m):
                    iw.append(pltpu.async_copy(src_hbm.at[wid, c0 + t],
                                               sidx[t], isem[t]))
                    iw.append(pltpu.async_copy(dst_hbm.at[wid, c0 + t],
                                               didx[t], isem[t]))
                gw = []
                for t in range(m):
                    iw[2 * t].wait()
                    gw.append(pltpu.async_copy(table_hbm.at[sidx[t]],
                                               rows[t], gsem[t]))
                sw = []
                for t in range(m):
                    gw[t].wait()
                    iw[2 * t + 1].wait()
                    sw.append(pltpu.async_copy(rows[t], agg_sh.at[didx[t]],
                                               ssem[t], add=True))
                for t in range(m):
                    sw[t].wait()

            def body(g, carry):
                group(g * gk, gk)
                return carry

            lax.fori_loop(0, ngroups, body, 0)
            if ntail:
                group(ngroups * gk, ntail)
        else:
            rows0 = bufs[0]
            didx = bufs[1:1 + gk]
            isem = bufs[1 + gk:1 + 2 * gk]
            ssem = bufs[1 + 2 * gk:]
            pltpu.sync_copy(ones_hbm, rows0)
            plsc.subcore_barrier()

            def group(c0, m):
                iw = []
                for t in range(m):
                    iw.append(pltpu.async_copy(dst_hbm.at[wid, c0 + t],
                                               didx[t], isem[t]))
                sw = []
                for t in range(m):
                    iw[t].wait()
                    sw.append(pltpu.async_copy(rows0, agg_sh.at[didx[t]],
                                               ssem[t], add=True))
                for t in range(m):
                    sw[t].wait()

            def body(g, carry):
                group(g * gk, gk)
                return carry

            lax.fori_loop(0, ngroups, body, 0)
            if ntail:
                group(ngroups * gk, ntail)

        plsc.subcore_barrier()
        _stripe_copy(sid, n, lambda o, s: agg_sh.at[pl.ds(o, s)],
                     lambda o, s: out_hbm.at[cid, pl.ds(o, s)])

    return k(src3, dst3, table, ones, zeros)


def _deg_dinv(degp_ref):
    deg = degp_ref[0, :, 0:1] + degp_ref[1, :, 0:1] + 1.0
    return deg, lax.rsqrt(deg)


def _tc_pre(x, W1, degp, *, n, d, r):
    def body(x_ref, w_ref, degp_ref, ts_ref, tself_ref):
        deg, dinv = _deg_dinv(degp_ref)
        t = jnp.dot(x_ref[...], w_ref[...], preferred_element_type=jnp.float32)
        ts_ref[...] = t * dinv
        tself_ref[...] = t / deg

    return pl.pallas_call(
        body,
        grid=(n // r,),
        in_specs=[
            pl.BlockSpec((r, d), lambda i: (i, 0)),
            pl.BlockSpec((d, d), lambda i: (0, 0)),
            pl.BlockSpec((NC, r, d), lambda i: (0, i, 0)),
        ],
        out_specs=[pl.BlockSpec((r, d), lambda i: (i, 0))] * 2,
        out_shape=[jax.ShapeDtypeStruct((n, d), jnp.float32)] * 2,
    )(x, W1, degp)


def _tc_mid(aggp, tself1, degp, b1, W2, *, n, d, r):
    def body(aggp_ref, tself_ref, degp_ref, b_ref, w_ref, ts_ref, tself2_ref):
        deg, dinv = _deg_dinv(degp_ref)
        agg = aggp_ref[0] + aggp_ref[1]
        h1 = jnp.maximum(agg * dinv + tself_ref[...] + b_ref[...], 0.0)
        t2 = jnp.dot(h1, w_ref[...], preferred_element_type=jnp.float32)
        ts_ref[...] = t2 * dinv
        tself2_ref[...] = t2 / deg

    return pl.pallas_call(
        body,
        grid=(n // r,),
        in_specs=[
            pl.BlockSpec((NC, r, d), lambda i: (0, i, 0)),
            pl.BlockSpec((r, d), lambda i: (i, 0)),
            pl.BlockSpec((NC, r, d), lambda i: (0, i, 0)),
            pl.BlockSpec((1, d), lambda i: (0, 0)),
            pl.BlockSpec((d, d), lambda i: (0, 0)),
        ],
        out_specs=[pl.BlockSpec((r, d), lambda i: (i, 0))] * 2,
        out_shape=[jax.ShapeDtypeStruct((n, d), jnp.float32)] * 2,
    )(aggp, tself1, degp, b1, W2)


def _tc_post(aggp, tself2, degp, b2, M1, bm1, M2, bm2, *, n, d, r):
    def body(aggp_ref, tself_ref, degp_ref, b2_ref, m1_ref, bm1_ref, m2_ref,
             bm2_ref, out_ref):
        _, dinv = _deg_dinv(degp_ref)
        agg = aggp_ref[0] + aggp_ref[1]
        h2 = agg * dinv + tself_ref[...] + b2_ref[...]
        h3 = jnp.maximum(
            jnp.dot(h2, m1_ref[...], preferred_element_type=jnp.float32)
            + bm1_ref[...], 0.0)
        out_ref[...] = jnp.dot(
            h3, m2_ref[...], preferred_element_type=jnp.float32) + bm2_ref[...]

    return pl.pallas_call(
        body,
        grid=(n // r,),
        in_specs=[
            pl.BlockSpec((NC, r, d), lambda i: (0, i, 0)),
            pl.BlockSpec((r, d), lambda i: (i, 0)),
            pl.BlockSpec((NC, r, d), lambda i: (0, i, 0)),
            pl.BlockSpec((1, d), lambda i: (0, 0)),
            pl.BlockSpec((d, d), lambda i: (0, 0)),
            pl.BlockSpec((1, d), lambda i: (0, 0)),
            pl.BlockSpec((d, d), lambda i: (0, 0)),
            pl.BlockSpec((1, d), lambda i: (0, 0)),
        ],
        out_specs=pl.BlockSpec((r, d), lambda i: (i, 0)),
        out_shape=jax.ShapeDtypeStruct((n, d), jnp.float32),
    )(aggp, tself2, degp, b2, M1, bm1, M2, bm2)


def kernel(x, edge_index, W1, b1, W2, b2, M1, bm1, M2, bm2):
    n, d = x.shape
    e = edge_index.shape[1]
    assert e % (NW * CHUNK) == 0 and n % NS == 0 and n % 8 == 0

    src = edge_index[0]
    dst = edge_index[1]
    zeros = jnp.zeros((n, d), jnp.float32)
    ones = jnp.ones((CHUNK, d), jnp.float32)
    r = 1000 if n % 1000 == 0 else 8

    degp = _edge_aggregate(src, dst, ones, ones, zeros, n=n, e=e, d=d,
                           gather=False)
    ts1, tself1 = _tc_pre(x, W1, degp, n=n, d=d, r=r)
    agg1 = _edge_aggregate(src, dst, ts1, ones, zeros, n=n, e=e, d=d)
    ts2, tself2 = _tc_mid(agg1, tself1, degp, b1.reshape(1, d), W2, n=n, d=d, r=r)
    agg2 = _edge_aggregate(src, dst, ts2, ones, zeros, n=n, e=e, d=d)
    return _tc_post(agg2, tself2, degp, b2.reshape(1, d), M1, bm1.reshape(1, d),
                    M2, bm2.reshape(1, d), n=n, d=d, r=r)


# speedup vs baseline: 1.0200x; 1.0200x over previous
import functools
import jax
import jax.numpy as jnp
from jax import lax
from jax.experimental import pallas as pl
from jax.experimental.pallas import tpu as pltpu
from jax.experimental.pallas import tpu_sc as plsc

NC = 2
NS = 16
NW = NC * NS
CHUNK = 100


def _sc_mesh():
    return plsc.VectorSubcoreMesh(core_axis_name="c", subcore_axis_name="s")


def _stripe_copy(sid, n, src_at, dst_at):
    r_small = (n // NS) // 8 * 8
    r_last = n - r_small * (NS - 1)

    @pl.when(sid < NS - 1)
    def _():
        pltpu.sync_copy(src_at(sid * r_small, r_small),
                        dst_at(sid * r_small, r_small))

    @pl.when(sid == NS - 1)
    def _():
        pltpu.sync_copy(src_at((NS - 1) * r_small, r_last),
                        dst_at((NS - 1) * r_small, r_last))


GK = 3
GKD = 8


@functools.partial(jax.jit, static_argnames=("n", "e", "d", "gather"))
def _edge_aggregate(src, dst, table, ones, zeros, *, n, e, d, gather=True):
    per_w = e // NW
    nchunks = per_w // CHUNK

    src3 = src.reshape(NW, nchunks, CHUNK)
    dst3 = dst.reshape(NW, nchunks, CHUNK)

    gk = GK if gather else GKD
    ngroups = nchunks // gk
    ntail = nchunks - ngroups * gk

    if gather:
        bufs_scratch = (
            [pltpu.VMEM((CHUNK, d), jnp.float32) for _ in range(gk)]
            + [pltpu.VMEM((CHUNK,), jnp.int32) for _ in range(2 * gk)]
            + [pltpu.SemaphoreType.DMA for _ in range(3 * gk)]
        )
    else:
        bufs_scratch = (
            [pltpu.VMEM((CHUNK, d), jnp.float32)]
            + [pltpu.VMEM((CHUNK,), jnp.int32) for _ in range(gk)]
            + [pltpu.SemaphoreType.DMA for _ in range(2 * gk)]
        )

    @functools.partial(
        pl.kernel,
        out_type=jax.ShapeDtypeStruct((NC, n, d), jnp.float32),
        mesh=_sc_mesh(),
        scratch_types=[pltpu.VMEM_SHARED((n, d), jnp.float32)] + bufs_scratch,
    )
    def k(src_hbm, dst_hbm, table_hbm, ones_hbm, z_hbm, out_hbm, agg_sh, *bufs):
        cid = lax.axis_index("c")
        sid = lax.axis_index("s")
        wid = cid * NS + sid
        _stripe_copy(sid, n, lambda o, s: z_hbm.at[pl.ds(o, s)],
                     lambda o, s: agg_sh.at[pl.ds(o, s)])

        if gather:
            rows = bufs[:gk]
            sidx = bufs[gk:2 * gk]
            didx = bufs[2 * gk:3 * gk]
            isem = bufs[3 * gk:4 * gk]
            gsem = bufs[4 * gk:5 * gk]
            ssem = bufs[5 * gk:6 * gk]
            plsc.subcore_barrier()

            def group(c0, m):
                iw = []
                for t in range(m):
                    iw.append(pltpu.async_copy(src_hbm.at[wid, c0 + t],
                                               sidx[t], isem[t]))
                    iw.append(pltpu.async_copy(dst_hbm.at[wid, c0 + t],
                                               didx[t], isem[t]))
                gw = []
                for t in range(m):
                    iw[2 * t].wait()
                    gw.append(pltpu.async_copy(table_hbm.at[sidx[t]],
                                               rows[t], gsem[t]))
                sw = []
                for t in range(m):
                    gw[t].wait()
                    iw[2 * t + 1].wait()
                    sw.append(pltpu.async_copy(rows[t], agg_sh.at[didx[t]],
                                               ssem[t], add=True))
                for t in range(m):
                    sw[t].wait()

            def body(g, carry):
                group(g * gk, gk)
                return carry

            lax.fori_loop(0, ngroups, body, 0)
            if ntail:
                group(ngroups * gk, ntail)
        else:
            rows0 = bufs[0]
            didx = bufs[1:1 + gk]
            isem = bufs[1 + gk:1 + 2 * gk]
            ssem = bufs[1 + 2 * gk:]
            pltpu.sync_copy(ones_hbm, rows0)
            plsc.subcore_barrier()

            def group(c0, m):
                iw = []
                for t in range(m):
                    iw.append(pltpu.async_copy(dst_hbm.at[wid, c0 + t],
                                               didx[t], isem[t]))
                sw = []
                for t in range(m):
                    iw[t].wait()
                    sw.append(pltpu.async_copy(rows0, agg_sh.at[didx[t]],
                                               ssem[t], add=True))
                for t in range(m):
                    sw[t].wait()

            def body(g, carry):
                group(g * gk, gk)
                return carry

            lax.fori_loop(0, ngroups, body, 0)
            if ntail:
                group(ngroups * gk, ntail)

        plsc.subcore_barrier()
        _stripe_copy(sid, n, lambda o, s: agg_sh.at[pl.ds(o, s)],
                     lambda o, s: out_hbm.at[cid, pl.ds(o, s)])

    return k(src3, dst3, table, ones, zeros)


def _deg_dinv(degp_ref):
    deg = degp_ref[0, :, 0:1] + degp_ref[1, :, 0:1] + 1.0
    return deg, lax.rsqrt(deg)


def _tc_mm(x, W, *, n, d, r):
    def body(x_ref, w_ref, t_ref):
        t_ref[...] = jnp.dot(x_ref[...], w_ref[...],
                             preferred_element_type=jnp.float32)

    return pl.pallas_call(
        body,
        grid=(n // r,),
        in_specs=[
            pl.BlockSpec((r, d), lambda i: (i, 0)),
            pl.BlockSpec((d, d), lambda i: (0, 0)),
        ],
        out_specs=pl.BlockSpec((r, d), lambda i: (i, 0)),
        out_shape=jax.ShapeDtypeStruct((n, d), jnp.float32),
    )(x, W)


def _tc_scale(t, degp, *, n, d, r):
    def body(t_ref, degp_ref, ts_ref, tself_ref):
        deg, dinv = _deg_dinv(degp_ref)
        t = t_ref[...]
        ts_ref[...] = t * dinv
        tself_ref[...] = t / deg

    return pl.pallas_call(
        body,
        grid=(n // r,),
        in_specs=[
            pl.BlockSpec((r, d), lambda i: (i, 0)),
            pl.BlockSpec((NC, r, d), lambda i: (0, i, 0)),
        ],
        out_specs=[pl.BlockSpec((r, d), lambda i: (i, 0))] * 2,
        out_shape=[jax.ShapeDtypeStruct((n, d), jnp.float32)] * 2,
    )(t, degp)


def _tc_mid(aggp, tself1, degp, b1, W2, *, n, d, r):
    def body(aggp_ref, tself_ref, degp_ref, b_ref, w_ref, ts_ref, tself2_ref):
        deg, dinv = _deg_dinv(degp_ref)
        agg = aggp_ref[0] + aggp_ref[1]
        h1 = jnp.maximum(agg * dinv + tself_ref[...] + b_ref[...], 0.0)
        t2 = jnp.dot(h1, w_ref[...], preferred_element_type=jnp.float32)
        ts_ref[...] = t2 * dinv
        tself2_ref[...] = t2 / deg

    return pl.pallas_call(
        body,
        grid=(n // r,),
        in_specs=[
            pl.BlockSpec((NC, r, d), lambda i: (0, i, 0)),
            pl.BlockSpec((r, d), lambda i: (i, 0)),
            pl.BlockSpec((NC, r, d), lambda i: (0, i, 0)),
            pl.BlockSpec((1, d), lambda i: (0, 0)),
            pl.BlockSpec((d, d), lambda i: (0, 0)),
        ],
        out_specs=[pl.BlockSpec((r, d), lambda i: (i, 0))] * 2,
        out_shape=[jax.ShapeDtypeStruct((n, d), jnp.float32)] * 2,
    )(aggp, tself1, degp, b1, W2)


def _tc_post(aggp, tself2, degp, b2, M1, bm1, M2, bm2, *, n, d, r):
    def body(aggp_ref, tself_ref, degp_ref, b2_ref, m1_ref, bm1_ref, m2_ref,
             bm2_ref, out_ref):
        _, dinv = _deg_dinv(degp_ref)
        agg = aggp_ref[0] + aggp_ref[1]
        h2 = agg * dinv + tself_ref[...] + b2_ref[...]
        h3 = jnp.maximum(
            jnp.dot(h2, m1_ref[...], preferred_element_type=jnp.float32)
            + bm1_ref[...], 0.0)
        out_ref[...] = jnp.dot(
            h3, m2_ref[...], preferred_element_type=jnp.float32) + bm2_ref[...]

    return pl.pallas_call(
        body,
        grid=(n // r,),
        in_specs=[
            pl.BlockSpec((NC, r, d), lambda i: (0, i, 0)),
            pl.BlockSpec((r, d), lambda i: (i, 0)),
            pl.BlockSpec((NC, r, d), lambda i: (0, i, 0)),
            pl.BlockSpec((1, d), lambda i: (0, 0)),
            pl.BlockSpec((d, d), lambda i: (0, 0)),
            pl.BlockSpec((1, d), lambda i: (0, 0)),
            pl.BlockSpec((d, d), lambda i: (0, 0)),
            pl.BlockSpec((1, d), lambda i: (0, 0)),
        ],
        out_specs=pl.BlockSpec((r, d), lambda i: (i, 0)),
        out_shape=jax.ShapeDtypeStruct((n, d), jnp.float32),
    )(aggp, tself2, degp, b2, M1, bm1, M2, bm2)


def kernel(x, edge_index, W1, b1, W2, b2, M1, bm1, M2, bm2):
    n, d = x.shape
    e = edge_index.shape[1]
    assert e % (NW * CHUNK) == 0 and n % NS == 0 and n % 8 == 0

    src = edge_index[0]
    dst = edge_index[1]
    zeros = jnp.zeros((n, d), jnp.float32)
    ones = jnp.ones((CHUNK, d), jnp.float32)
    r = 1000 if n % 1000 == 0 else 8

    t1 = _tc_mm(x, W1, n=n, d=d, r=r)
    degp = _edge_aggregate(src, dst, ones, ones, zeros, n=n, e=e, d=d,
                           gather=False)
    ts1, tself1 = _tc_scale(t1, degp, n=n, d=d, r=r)
    agg1 = _edge_aggregate(src, dst, ts1, ones, zeros, n=n, e=e, d=d)
    ts2, tself2 = _tc_mid(agg1, tself1, degp, b1.reshape(1, d), W2, n=n, d=d, r=r)
    agg2 = _edge_aggregate(src, dst, ts2, ones, zeros, n=n, e=e, d=d)
    return _tc_post(agg2, tself2, degp, b2.reshape(1, d), M1, bm1.reshape(1, d),
                    M2, bm2.reshape(1, d), n=n, d=d, r=r)


# speedup vs baseline: 1.0333x; 1.0131x over previous
import functools
import jax
import jax.numpy as jnp
from jax import lax
from jax.experimental import pallas as pl
from jax.experimental.pallas import tpu as pltpu
from jax.experimental.pallas import tpu_sc as plsc

NC = 2
NS = 16
NW = NC * NS
CHUNK = 100


def _sc_mesh():
    return plsc.VectorSubcoreMesh(core_axis_name="c", subcore_axis_name="s")


def _stripe_copy(sid, n, src_at, dst_at):
    r_small = (n // NS) // 8 * 8
    r_last = n - r_small * (NS - 1)

    @pl.when(sid < NS - 1)
    def _():
        pltpu.sync_copy(src_at(sid * r_small, r_small),
                        dst_at(sid * r_small, r_small))

    @pl.when(sid == NS - 1)
    def _():
        pltpu.sync_copy(src_at((NS - 1) * r_small, r_last),
                        dst_at((NS - 1) * r_small, r_last))


GK = 3
GKD = 8


@functools.partial(jax.jit, static_argnames=("n", "e", "d", "gather"))
def _edge_aggregate(src, dst, table, ones, zeros, *, n, e, d, gather=True):
    per_w = e // NW
    nchunks = per_w // CHUNK

    src3 = src.reshape(NW, nchunks, CHUNK)
    dst3 = dst.reshape(NW, nchunks, CHUNK)

    gk = GK if gather else GKD
    ngroups = nchunks // gk
    ntail = nchunks - ngroups * gk

    if gather:
        bufs_scratch = (
            [pltpu.VMEM((CHUNK, d), jnp.float32) for _ in range(gk)]
            + [pltpu.VMEM((CHUNK,), jnp.int32) for _ in range(2 * gk)]
            + [pltpu.SemaphoreType.DMA for _ in range(3 * gk)]
        )
    else:
        bufs_scratch = (
            [pltpu.VMEM((CHUNK, d), jnp.float32)]
            + [pltpu.VMEM((CHUNK,), jnp.int32) for _ in range(gk)]
            + [pltpu.SemaphoreType.DMA for _ in range(2 * gk)]
        )

    @functools.partial(
        pl.kernel,
        out_type=jax.ShapeDtypeStruct((NC, n, d), jnp.float32),
        mesh=_sc_mesh(),
        scratch_types=[pltpu.VMEM_SHARED((n, d), jnp.float32)] + bufs_scratch,
    )
    def k(src_hbm, dst_hbm, table_hbm, ones_hbm, z_hbm, out_hbm, agg_sh, *bufs):
        cid = lax.axis_index("c")
        sid = lax.axis_index("s")
        wid = cid * NS + sid
        _stripe_copy(sid, n, lambda o, s: z_hbm.at[pl.ds(o, s)],
                     lambda o, s: agg_sh.at[pl.ds(o, s)])

        if gather:
            rows = bufs[:gk]
            sidx = bufs[gk:2 * gk]
            didx = bufs[2 * gk:3 * gk]
            isem = bufs[3 * gk:4 * gk]
            gsem = bufs[4 * gk:5 * gk]
            ssem = bufs[5 * gk:6 * gk]
            plsc.subcore_barrier()

            def group(c0, m):
                iw = []
                for t in range(m):
                    iw.append(pltpu.async_copy(src_hbm.at[wid, c0 + t],
                                               sidx[t], isem[t]))
                    iw.append(pltpu.async_copy(dst_hbm.at[wid, c0 + t],
                                               didx[t], isem[t]))
                gw = []
                for t in range(m):
                    iw[2 * t].wait()
                    gw.append(pltpu.async_copy(table_hbm.at[sidx[t]],
                                               rows[t], gsem[t]))
                sw = []
                for t in range(m):
                    gw[t].wait()
                    iw[2 * t + 1].wait()
                    sw.append(pltpu.async_copy(rows[t], agg_sh.at[didx[t]],
                                               ssem[t], add=True))
                for t in range(m):
                    sw[t].wait()

            def body(g, carry):
                group(g * gk, gk)
                return carry

            lax.fori_loop(0, ngroups, body, 0)
            if ntail:
                group(ngroups * gk, ntail)
        else:
            rows0 = bufs[0]
            didx = bufs[1:1 + gk]
            isem = bufs[1 + gk:1 + 2 * gk]
            ssem = bufs[1 + 2 * gk:]
            pltpu.sync_copy(ones_hbm, rows0)
            plsc.subcore_barrier()

            def group(c0, m):
                iw = []
                for t in range(m):
                    iw.append(pltpu.async_copy(dst_hbm.at[wid, c0 + t],
                                               didx[t], isem[t]))
                sw = []
                for t in range(m):
                    iw[t].wait()
                    sw.append(pltpu.async_copy(rows0, agg_sh.at[didx[t]],
                                               ssem[t], add=True))
                for t in range(m):
                    sw[t].wait()

            def body(g, carry):
                group(g * gk, gk)
                return carry

            lax.fori_loop(0, ngroups, body, 0)
            if ntail:
                group(ngroups * gk, ntail)

        plsc.subcore_barrier()
        _stripe_copy(sid, n, lambda o, s: agg_sh.at[pl.ds(o, s)],
                     lambda o, s: out_hbm.at[cid, pl.ds(o, s)])

    return k(src3, dst3, table, ones, zeros)


def _deg_dinv(degp_ref):
    deg = degp_ref[0, :, 0:1] + degp_ref[1, :, 0:1] + 1.0
    return deg, lax.rsqrt(deg)


def _tc_mm(x, W, *, n, d, r):
    def body(x_ref, w_ref, t_ref):
        t_ref[...] = jnp.dot(x_ref[...], w_ref[...],
                             preferred_element_type=jnp.float32)

    return pl.pallas_call(
        body,
        grid=(n // r,),
        in_specs=[
            pl.BlockSpec((r, d), lambda i: (i, 0)),
            pl.BlockSpec((d, d), lambda i: (0, 0)),
        ],
        out_specs=pl.BlockSpec((r, d), lambda i: (i, 0)),
        out_shape=jax.ShapeDtypeStruct((n, d), jnp.float32),
    )(x, W)


def _tc_scale(t, degp, *, n, d, r):
    def body(t_ref, degp_ref, ts_ref, tself_ref):
        deg, dinv = _deg_dinv(degp_ref)
        t = t_ref[...]
        ts_ref[...] = t * dinv
        tself_ref[...] = t / deg

    return pl.pallas_call(
        body,
        grid=(n // r,),
        in_specs=[
            pl.BlockSpec((r, d), lambda i: (i, 0)),
            pl.BlockSpec((NC, r, d), lambda i: (0, i, 0)),
        ],
        out_specs=[pl.BlockSpec((r, d), lambda i: (i, 0))] * 2,
        out_shape=[jax.ShapeDtypeStruct((n, d), jnp.float32)] * 2,
    )(t, degp)


def _tc_mid(aggp, tself1, degp, b1, W2, *, n, d, r):
    def body(aggp_ref, tself_ref, degp_ref, b_ref, w_ref, ts_ref, tself2_ref):
        deg, dinv = _deg_dinv(degp_ref)
        agg = aggp_ref[0] + aggp_ref[1]
        h1 = jnp.maximum(agg * dinv + tself_ref[...] + b_ref[...], 0.0)
        t2 = jnp.dot(h1, w_ref[...], preferred_element_type=jnp.float32)
        ts_ref[...] = t2 * dinv
        tself2_ref[...] = t2 / deg

    return pl.pallas_call(
        body,
        grid=(n // r,),
        in_specs=[
            pl.BlockSpec((NC, r, d), lambda i: (0, i, 0)),
            pl.BlockSpec((r, d), lambda i: (i, 0)),
            pl.BlockSpec((NC, r, d), lambda i: (0, i, 0)),
            pl.BlockSpec((1, d), lambda i: (0, 0)),
            pl.BlockSpec((d, d), lambda i: (0, 0)),
        ],
        out_specs=[pl.BlockSpec((r, d), lambda i: (i, 0))] * 2,
        out_shape=[jax.ShapeDtypeStruct((n, d), jnp.float32)] * 2,
    )(aggp, tself1, degp, b1, W2)


def _tc_post(aggp, tself2, degp, b2, M1, bm1, M2, bm2, *, n, d, r):
    def body(aggp_ref, tself_ref, degp_ref, b2_ref, m1_ref, bm1_ref, m2_ref,
             bm2_ref, out_ref):
        _, dinv = _deg_dinv(degp_ref)
        agg = aggp_ref[0] + aggp_ref[1]
        h2 = agg * dinv + tself_ref[...] + b2_ref[...]
        h3 = jnp.maximum(
            jnp.dot(h2, m1_ref[...], preferred_element_type=jnp.float32)
            + bm1_ref[...], 0.0)
        out_ref[...] = jnp.dot(
            h3, m2_ref[...], preferred_element_type=jnp.float32) + bm2_ref[...]

    return pl.pallas_call(
        body,
        grid=(n // r,),
        in_specs=[
            pl.BlockSpec((NC, r, d), lambda i: (0, i, 0)),
            pl.BlockSpec((r, d), lambda i: (i, 0)),
            pl.BlockSpec((NC, r, d), lambda i: (0, i, 0)),
            pl.BlockSpec((1, d), lambda i: (0, 0)),
            pl.BlockSpec((d, d), lambda i: (0, 0)),
            pl.BlockSpec((1, d), lambda i: (0, 0)),
            pl.BlockSpec((d, d), lambda i: (0, 0)),
            pl.BlockSpec((1, d), lambda i: (0, 0)),
        ],
        out_specs=pl.BlockSpec((r, d), lambda i: (i, 0)),
        out_shape=jax.ShapeDtypeStruct((n, d), jnp.float32),
    )(aggp, tself2, degp, b2, M1, bm1, M2, bm2)


def kernel(x, edge_index, W1, b1, W2, b2, M1, bm1, M2, bm2):
    n, d = x.shape
    e = edge_index.shape[1]
    assert e % (NW * CHUNK) == 0 and n % NS == 0 and n % 8 == 0

    src = edge_index[0]
    dst = edge_index[1]
    zeros = jnp.zeros((n, d), jnp.float32)
    ones = jnp.ones((CHUNK, d), jnp.float32)
    r = 2000 if n % 2000 == 0 else 8

    t1 = _tc_mm(x, W1, n=n, d=d, r=r)
    degp = _edge_aggregate(src, dst, ones, ones, zeros, n=n, e=e, d=d,
                           gather=False)
    ts1, tself1 = _tc_scale(t1, degp, n=n, d=d, r=r)
    agg1 = _edge_aggregate(src, dst, ts1, ones, zeros, n=n, e=e, d=d)
    ts2, tself2 = _tc_mid(agg1, tself1, degp, b1.reshape(1, d), W2, n=n, d=d, r=r)
    agg2 = _edge_aggregate(src, dst, ts2, ones, zeros, n=n, e=e, d=d)
    return _tc_post(agg2, tself2, degp, b2.reshape(1, d), M1, bm1.reshape(1, d),
                    M2, bm2.reshape(1, d), n=n, d=d, r=r)


# speedup vs baseline: 1.0356x; 1.0022x over previous
import functools
import jax
import jax.numpy as jnp
from jax import lax
from jax.experimental import pallas as pl
from jax.experimental.pallas import tpu as pltpu
from jax.experimental.pallas import tpu_sc as plsc

NC = 2
NS = 16
NW = NC * NS
CHUNK = 100


def _sc_mesh():
    return plsc.VectorSubcoreMesh(core_axis_name="c", subcore_axis_name="s")


def _stripe_copy(sid, n, src_at, dst_at):
    r_small = (n // NS) // 8 * 8
    r_last = n - r_small * (NS - 1)

    @pl.when(sid < NS - 1)
    def _():
        pltpu.sync_copy(src_at(sid * r_small, r_small),
                        dst_at(sid * r_small, r_small))

    @pl.when(sid == NS - 1)
    def _():
        pltpu.sync_copy(src_at((NS - 1) * r_small, r_last),
                        dst_at((NS - 1) * r_small, r_last))


GK = 3
GKD = 8


@functools.partial(jax.jit, static_argnames=("n", "e", "d", "gather"))
def _edge_aggregate(src, dst, table, ones, zeros, *, n, e, d, gather=True):
    per_w = e // NW
    nchunks = per_w // CHUNK

    src3 = src.reshape(NW, nchunks, CHUNK)
    dst3 = dst.reshape(NW, nchunks, CHUNK)

    gk = GK if gather else GKD
    ngroups = nchunks // gk
    ntail = nchunks - ngroups * gk

    if gather:
        bufs_scratch = (
            [pltpu.VMEM((CHUNK, d), jnp.float32) for _ in range(gk)]
            + [pltpu.VMEM((CHUNK,), jnp.int32) for _ in range(2 * gk)]
            + [pltpu.SemaphoreType.DMA for _ in range(3 * gk)]
        )
    else:
        bufs_scratch = (
            [pltpu.VMEM((CHUNK, d), jnp.float32)]
            + [pltpu.VMEM((CHUNK,), jnp.int32) for _ in range(gk)]
            + [pltpu.SemaphoreType.DMA for _ in range(2 * gk)]
        )

    @functools.partial(
        pl.kernel,
        out_type=jax.ShapeDtypeStruct((NC, n, d), jnp.float32),
        mesh=_sc_mesh(),
        scratch_types=[pltpu.VMEM_SHARED((n, d), jnp.float32)] + bufs_scratch,
    )
    def k(src_hbm, dst_hbm, table_hbm, ones_hbm, z_hbm, out_hbm, agg_sh, *bufs):
        cid = lax.axis_index("c")
        sid = lax.axis_index("s")
        wid = cid * NS + sid
        _stripe_copy(sid, n, lambda o, s: z_hbm.at[pl.ds(o, s)],
                     lambda o, s: agg_sh.at[pl.ds(o, s)])

        if gather:
            rows = bufs[:gk]
            sidx = bufs[gk:2 * gk]
            didx = bufs[2 * gk:3 * gk]
            isem = bufs[3 * gk:4 * gk]
            gsem = bufs[4 * gk:5 * gk]
            ssem = bufs[5 * gk:6 * gk]
            plsc.subcore_barrier()

            def group(c0, m):
                iw = []
                for t in range(m):
                    iw.append(pltpu.async_copy(src_hbm.at[wid, c0 + t],
                                               sidx[t], isem[t]))
                    iw.append(pltpu.async_copy(dst_hbm.at[wid, c0 + t],
                                               didx[t], isem[t]))
                gw = []
                for t in range(m):
                    iw[2 * t].wait()
                    gw.append(pltpu.async_copy(table_hbm.at[sidx[t]],
                                               rows[t], gsem[t]))
                sw = []
                for t in range(m):
                    gw[t].wait()
                    iw[2 * t + 1].wait()
                    sw.append(pltpu.async_copy(rows[t], agg_sh.at[didx[t]],
                                               ssem[t], add=True))
                for t in range(m):
                    sw[t].wait()

            def body(g, carry):
                group(g * gk, gk)
                return carry

            lax.fori_loop(0, ngroups, body, 0)
            if ntail:
                group(ngroups * gk, ntail)
        else:
            rows0 = bufs[0]
            didx = bufs[1:1 + gk]
            isem = bufs[1 + gk:1 + 2 * gk]
            ssem = bufs[1 + 2 * gk:]
            pltpu.sync_copy(ones_hbm, rows0)
            plsc.subcore_barrier()

            def group(c0, m):
                iw = []
                for t in range(m):
                    iw.append(pltpu.async_copy(dst_hbm.at[wid, c0 + t],
                                               didx[t], isem[t]))
                sw = []
                for t in range(m):
                    iw[t].wait()
                    sw.append(pltpu.async_copy(rows0, agg_sh.at[didx[t]],
                                               ssem[t], add=True))
                for t in range(m):
                    sw[t].wait()

            def body(g, carry):
                group(g * gk, gk)
                return carry

            lax.fori_loop(0, ngroups, body, 0)
            if ntail:
                group(ngroups * gk, ntail)

        plsc.subcore_barrier()
        _stripe_copy(sid, n, lambda o, s: agg_sh.at[pl.ds(o, s)],
                     lambda o, s: out_hbm.at[cid, pl.ds(o, s)])

    return k(src3, dst3, table, ones, zeros)


def _deg_dinv(degp_ref):
    deg = degp_ref[0, :, 0:1] + degp_ref[1, :, 0:1] + 1.0
    return deg, lax.rsqrt(deg)


def _tc_mm(x, W, *, n, d, r):
    def body(x_ref, w_ref, t_ref):
        t_ref[...] = jnp.dot(x_ref[...], w_ref[...],
                             preferred_element_type=jnp.float32)

    return pl.pallas_call(
        body,
        grid=(n // r,),
        in_specs=[
            pl.BlockSpec((r, d), lambda i: (i, 0)),
            pl.BlockSpec((d, d), lambda i: (0, 0)),
        ],
        out_specs=pl.BlockSpec((r, d), lambda i: (i, 0)),
        out_shape=jax.ShapeDtypeStruct((n, d), jnp.float32),
    )(x, W)


def _tc_scale(t, degp, *, n, d, r):
    def body(t_ref, degp_ref, ts_ref, tself_ref):
        deg, dinv = _deg_dinv(degp_ref)
        t = t_ref[...]
        ts_ref[...] = t * dinv
        tself_ref[...] = t / deg

    return pl.pallas_call(
        body,
        grid=(n // r,),
        in_specs=[
            pl.BlockSpec((r, d), lambda i: (i, 0)),
            pl.BlockSpec((NC, r, d), lambda i: (0, i, 0)),
        ],
        out_specs=[pl.BlockSpec((r, d), lambda i: (i, 0))] * 2,
        out_shape=[jax.ShapeDtypeStruct((n, d), jnp.float32)] * 2,
    )(t, degp)


def _tc_mid(aggp, tself1, degp, b1, W2, *, n, d, r):
    def body(aggp_ref, tself_ref, degp_ref, b_ref, w_ref, ts_ref, tself2_ref):
        deg, dinv = _deg_dinv(degp_ref)
        agg = aggp_ref[0] + aggp_ref[1]
        h1 = jnp.maximum(agg * dinv + tself_ref[...] + b_ref[...], 0.0)
        t2 = jnp.dot(h1, w_ref[...], preferred_element_type=jnp.float32)
        ts_ref[...] = t2 * dinv
        tself2_ref[...] = t2 / deg

    return pl.pallas_call(
        body,
        grid=(n // r,),
        in_specs=[
            pl.BlockSpec((NC, r, d), lambda i: (0, i, 0)),
            pl.BlockSpec((r, d), lambda i: (i, 0)),
            pl.BlockSpec((NC, r, d), lambda i: (0, i, 0)),
            pl.BlockSpec((1, d), lambda i: (0, 0)),
            pl.BlockSpec((d, d), lambda i: (0, 0)),
        ],
        out_specs=[pl.BlockSpec((r, d), lambda i: (i, 0))] * 2,
        out_shape=[jax.ShapeDtypeStruct((n, d), jnp.float32)] * 2,
    )(aggp, tself1, degp, b1, W2)


def _tc_post(aggp, tself2, degp, b2, M1, bm1, M2, bm2, *, n, d, r):
    def body(aggp_ref, tself_ref, degp_ref, b2_ref, m1_ref, bm1_ref, m2_ref,
             bm2_ref, out_ref):
        _, dinv = _deg_dinv(degp_ref)
        agg = aggp_ref[0] + aggp_ref[1]
        h2 = agg * dinv + tself_ref[...] + b2_ref[...]
        h3 = jnp.maximum(
            jnp.dot(h2, m1_ref[...], preferred_element_type=jnp.float32)
            + bm1_ref[...], 0.0)
        out_ref[...] = jnp.dot(
            h3, m2_ref[...], preferred_element_type=jnp.float32) + bm2_ref[...]

    return pl.pallas_call(
        body,
        grid=(n // r,),
        in_specs=[
            pl.BlockSpec((NC, r, d), lambda i: (0, i, 0)),
            pl.BlockSpec((r, d), lambda i: (i, 0)),
            pl.BlockSpec((NC, r, d), lambda i: (0, i, 0)),
            pl.BlockSpec((1, d), lambda i: (0, 0)),
            pl.BlockSpec((d, d), lambda i: (0, 0)),
            pl.BlockSpec((1, d), lambda i: (0, 0)),
            pl.BlockSpec((d, d), lambda i: (0, 0)),
            pl.BlockSpec((1, d), lambda i: (0, 0)),
        ],
        out_specs=pl.BlockSpec((r, d), lambda i: (i, 0)),
        out_shape=jax.ShapeDtypeStruct((n, d), jnp.float32),
    )(aggp, tself2, degp, b2, M1, bm1, M2, bm2)


def kernel(x, edge_index, W1, b1, W2, b2, M1, bm1, M2, bm2):
    n, d = x.shape
    e = edge_index.shape[1]
    assert e % (NW * CHUNK) == 0 and n % NS == 0 and n % 8 == 0

    src = edge_index[0]
    dst = edge_index[1]
    zeros = jnp.zeros((n, d), jnp.float32)
    ones = jnp.ones((CHUNK, d), jnp.float32)
    r = 5000 if n % 5000 == 0 else 8

    t1 = _tc_mm(x, W1, n=n, d=d, r=r)
    degp = _edge_aggregate(src, dst, ones, ones, zeros, n=n, e=e, d=d,
                           gather=False)
    ts1, tself1 = _tc_scale(t1, degp, n=n, d=d, r=r)
    agg1 = _edge_aggregate(src, dst, ts1, ones, zeros, n=n, e=e, d=d)
    ts2, tself2 = _tc_mid(agg1, tself1, degp, b1.reshape(1, d), W2, n=n, d=d, r=r)
    agg2 = _edge_aggregate(src, dst, ts2, ones, zeros, n=n, e=e, d=d)
    return _tc_post(agg2, tself2, degp, b2.reshape(1, d), M1, bm1.reshape(1, d),
                    M2, bm2.reshape(1, d), n=n, d=d, r=r)
